# baseline (device time: 16785 ns/iter reference)
import jax
import jax.numpy as jnp
from jax import lax
from jax.experimental import pallas as pl
from jax.experimental.pallas import tpu as pltpu

_NDEV = 8


def kernel(x, dy, gamma):
    del gamma
    m, d = x.shape
    rows = m // 4
    half = rows // 2

    def body(x_hbm, dy_hbm, out_ref, xbuf, dybuf, gather_ref,
             local_sems, send_sems, recv_sems):
        my_x = lax.axis_index("x")
        my_y = lax.axis_index("y")
        my_z = lax.axis_index("z")
        my_id = my_x * 4 + my_y * 2 + my_z
        start = (2 * my_x + my_z) * rows

        cps = []
        for c in range(2):
            cpx = pltpu.make_async_copy(
                x_hbm.at[pl.ds(start + c * half, half), :],
                xbuf.at[c], local_sems.at[2 * c])
            cpd = pltpu.make_async_copy(
                dy_hbm.at[pl.ds(start + c * half, half), :],
                dybuf.at[c], local_sems.at[2 * c + 1])
            cpx.start()
            cpd.start()
            cps.append((cpx, cpd))

        barrier_sem = pltpu.get_barrier_semaphore()
        peers = []
        for mask in range(1, _NDEV):
            peer = (my_x ^ (mask >> 2), my_y ^ ((mask >> 1) & 1),
                    my_z ^ (mask & 1))
            peers.append(peer)
            pl.semaphore_signal(
                barrier_sem, inc=1, device_id=peer,
                device_id_type=pl.DeviceIdType.MESH,
            )

        rdmas = []

        def scatter(rnd):
            for k, peer in enumerate(peers):
                rdma = pltpu.make_async_remote_copy(
                    src_ref=gather_ref.at[rnd, my_id],
                    dst_ref=gather_ref.at[rnd, my_id],
                    send_sem=send_sems.at[rnd * (_NDEV - 1) + k],
                    recv_sem=recv_sems.at[rnd * (_NDEV - 1) + k],
                    device_id=peer,
                    device_id_type=pl.DeviceIdType.MESH,
                )
                rdma.start()
                rdmas.append(rdma)

        for c in range(2):
            cps[c][0].wait()
            cps[c][1].wait()
            xv = xbuf[c]
            dyv = dybuf[c]
            mu = jnp.mean(xv, axis=1, keepdims=True)
            var = jnp.mean(xv * xv, axis=1, keepdims=True) - mu * mu
            rstd = lax.rsqrt(var + 1e-5)
            xhat = (xv - mu) * rstd
            gather_ref[c, my_id, 0, :] = jnp.sum(dyv * xhat, axis=0)
            gather_ref[c, my_id, 1, :] = jnp.sum(dyv, axis=0)
            if c == 0:
                pl.semaphore_wait(barrier_sem, _NDEV - 1)
            scatter(c)

        for rdma in rdmas:
            rdma.wait()

        out_ref[:, :] = jnp.sum(gather_ref[:, :, :, :], axis=(0, 1))

    return pl.pallas_call(
        body,
        out_shape=jax.ShapeDtypeStruct((2, d), jnp.float32),
        in_specs=[
            pl.BlockSpec(memory_space=pltpu.MemorySpace.HBM),
            pl.BlockSpec(memory_space=pltpu.MemorySpace.HBM),
        ],
        out_specs=pl.BlockSpec(memory_space=pltpu.VMEM),
        scratch_shapes=[
            pltpu.VMEM((2, half, d), jnp.float32),
            pltpu.VMEM((2, half, d), jnp.float32),
            pltpu.VMEM((2, _NDEV, 2, d), jnp.float32),
            pltpu.SemaphoreType.DMA((4,)),
            pltpu.SemaphoreType.DMA((2 * (_NDEV - 1),)),
            pltpu.SemaphoreType.DMA((2 * (_NDEV - 1),)),
        ],
        compiler_params=pltpu.CompilerParams(collective_id=0),
    )(x, dy)


# device time: 11887 ns/iter; 1.4120x vs baseline; 1.4120x over previous
import jax
import jax.numpy as jnp
from jax import lax
from jax.experimental import pallas as pl
from jax.experimental.pallas import tpu as pltpu

_NDEV = 8


def kernel(x, dy, gamma):
    del gamma
    m, d = x.shape
    rows = m // 4
    half = rows // 2

    def body(x_hbm, dy_hbm, out_ref, xbuf, dybuf, gather_ref,
             local_sems, send_sems, recv_sems):
        my_x = lax.axis_index("x")
        my_y = lax.axis_index("y")
        my_z = lax.axis_index("z")
        my_id = my_x * 4 + my_y * 2 + my_z
        start = (2 * my_x + my_z) * rows

        cps = []
        for c in range(2):
            cpx = pltpu.make_async_copy(
                x_hbm.at[pl.ds(start + c * half, half), :],
                xbuf.at[c], local_sems.at[2 * c])
            cpd = pltpu.make_async_copy(
                dy_hbm.at[pl.ds(start + c * half, half), :],
                dybuf.at[c], local_sems.at[2 * c + 1])
            cpx.start()
            cpd.start()
            cps.append((cpx, cpd))

        barrier_sem = pltpu.get_barrier_semaphore()
        peers = []
        for mask in range(1, _NDEV):
            peer = (my_x ^ (mask >> 2), my_y ^ ((mask >> 1) & 1),
                    my_z ^ (mask & 1))
            peers.append(peer)
            pl.semaphore_signal(
                barrier_sem, inc=1, device_id=peer,
                device_id_type=pl.DeviceIdType.MESH,
            )

        rdmas = []

        def scatter(rnd):
            for k, peer in enumerate(peers):
                rdma = pltpu.make_async_remote_copy(
                    src_ref=gather_ref.at[rnd, my_id],
                    dst_ref=gather_ref.at[rnd, my_id],
                    send_sem=send_sems.at[rnd * (_NDEV - 1) + k],
                    recv_sem=recv_sems.at[rnd * (_NDEV - 1) + k],
                    device_id=peer,
                    device_id_type=pl.DeviceIdType.MESH,
                )
                rdma.start()
                rdmas.append(rdma)

        for c in range(2):
            cps[c][0].wait()
            cps[c][1].wait()
            xv = xbuf[c]
            dyv = dybuf[c]
            mu = jnp.mean(xv, axis=1, keepdims=True)
            var = jnp.mean(xv * xv, axis=1, keepdims=True) - mu * mu
            rstd = lax.rsqrt(var + 1e-5)
            xhat = (xv - mu) * rstd
            gather_ref[c, my_id, 0, :] = jnp.sum(dyv * xhat, axis=0)
            gather_ref[c, my_id, 1, :] = jnp.sum(dyv, axis=0)
            if c == 0:
                pl.semaphore_wait(barrier_sem, _NDEV - 1)
            scatter(c)

        for rdma in rdmas:
            rdma.wait()

        out_ref[:, :] = jnp.sum(gather_ref[:, :, :, :], axis=(0, 1))

    return pl.pallas_call(
        body,
        out_shape=jax.ShapeDtypeStruct((2, d), jnp.float32),
        in_specs=[
            pl.BlockSpec(memory_space=pltpu.MemorySpace.HBM),
            pl.BlockSpec(memory_space=pltpu.MemorySpace.HBM),
        ],
        out_specs=pl.BlockSpec(memory_space=pltpu.VMEM),
        scratch_shapes=[
            pltpu.VMEM((2, half, d), jnp.float32),
            pltpu.VMEM((2, half, d), jnp.float32),
            pltpu.VMEM((2, _NDEV, 2, d), jnp.float32),
            pltpu.SemaphoreType.DMA((4,)),
            pltpu.SemaphoreType.DMA((2 * (_NDEV - 1),)),
            pltpu.SemaphoreType.DMA((2 * (_NDEV - 1),)),
        ],
        compiler_params=pltpu.CompilerParams(collective_id=0),
    )(
        pltpu.with_memory_space_constraint(x, pltpu.MemorySpace.HBM),
        pltpu.with_memory_space_constraint(dy, pltpu.MemorySpace.HBM),
    )


# device time: 11691 ns/iter; 1.4357x vs baseline; 1.0168x over previous
import jax
import jax.numpy as jnp
from jax import lax
from jax.experimental import pallas as pl
from jax.experimental.pallas import tpu as pltpu

_NDEV = 8


def kernel(x, dy, gamma):
    del gamma
    m, d = x.shape
    rows = m // 4
    half = rows // 2

    def body(x_hbm, dy_hbm, out_ref, xbuf, dybuf, gather_ref,
             local_sems, send_sems, recv_sems):
        my_x = lax.axis_index("x")
        my_y = lax.axis_index("y")
        my_z = lax.axis_index("z")
        my_id = my_x * 4 + my_y * 2 + my_z
        start = (2 * my_x + my_z) * rows

        cps = []
        for c in range(2):
            cpx = pltpu.make_async_copy(
                x_hbm.at[pl.ds(start + c * half, half), :],
                xbuf.at[c], local_sems.at[2 * c])
            cpd = pltpu.make_async_copy(
                dy_hbm.at[pl.ds(start + c * half, half), :],
                dybuf.at[c], local_sems.at[2 * c + 1])
            cpx.start()
            cpd.start()
            cps.append((cpx, cpd))

        barrier_sem = pltpu.get_barrier_semaphore()
        peers = []
        for mask in range(1, _NDEV):
            peer = (my_x ^ (mask >> 2), my_y ^ ((mask >> 1) & 1),
                    my_z ^ (mask & 1))
            peers.append(peer)
            pl.semaphore_signal(
                barrier_sem, inc=1, device_id=peer,
                device_id_type=pl.DeviceIdType.MESH,
            )

        pdg = None
        pdb = None
        for c in range(2):
            cps[c][0].wait()
            cps[c][1].wait()
            xv = xbuf[c]
            dyv = dybuf[c]
            mu = jnp.mean(xv, axis=1, keepdims=True)
            var = jnp.mean(xv * xv, axis=1, keepdims=True) - mu * mu
            rstd = lax.rsqrt(var + 1e-5)
            xhat = (xv - mu) * rstd
            g = jnp.sum(dyv * xhat, axis=0)
            b = jnp.sum(dyv, axis=0)
            pdg = g if pdg is None else pdg + g
            pdb = b if pdb is None else pdb + b
        gather_ref[my_id, 0, :] = pdg
        gather_ref[my_id, 1, :] = pdb

        pl.semaphore_wait(barrier_sem, _NDEV - 1)

        rdmas = []
        for k, peer in enumerate(peers):
            rdma = pltpu.make_async_remote_copy(
                src_ref=gather_ref.at[my_id],
                dst_ref=gather_ref.at[my_id],
                send_sem=send_sems.at[k],
                recv_sem=recv_sems.at[k],
                device_id=peer,
                device_id_type=pl.DeviceIdType.MESH,
            )
            rdma.start()
            rdmas.append(rdma)
        for rdma in rdmas:
            rdma.wait()

        out_ref[:, :] = jnp.sum(gather_ref[:, :, :], axis=0)

    return pl.pallas_call(
        body,
        out_shape=jax.ShapeDtypeStruct((2, d), jnp.float32),
        in_specs=[
            pl.BlockSpec(memory_space=pltpu.MemorySpace.HBM),
            pl.BlockSpec(memory_space=pltpu.MemorySpace.HBM),
        ],
        out_specs=pl.BlockSpec(memory_space=pltpu.VMEM),
        scratch_shapes=[
            pltpu.VMEM((2, half, d), jnp.float32),
            pltpu.VMEM((2, half, d), jnp.float32),
            pltpu.VMEM((_NDEV, 2, d), jnp.float32),
            pltpu.SemaphoreType.DMA((4,)),
            pltpu.SemaphoreType.DMA((_NDEV - 1,)),
            pltpu.SemaphoreType.DMA((_NDEV - 1,)),
        ],
        compiler_params=pltpu.CompilerParams(collective_id=0),
    )(
        pltpu.with_memory_space_constraint(x, pltpu.MemorySpace.HBM),
        pltpu.with_memory_space_constraint(dy, pltpu.MemorySpace.HBM),
    )


# device time: 11201 ns/iter; 1.4985x vs baseline; 1.0437x over previous
import jax
import jax.numpy as jnp
from jax import lax
from jax.experimental import pallas as pl
from jax.experimental.pallas import tpu as pltpu

_NDEV = 8


def kernel(x, dy, gamma):
    del gamma
    m, d = x.shape
    rows = m // 4
    n_chunks = 4
    half = rows // n_chunks

    def body(x_hbm, dy_hbm, out_ref, xbuf, dybuf, gather_ref,
             local_sems, send_sems, recv_sems):
        my_x = lax.axis_index("x")
        my_y = lax.axis_index("y")
        my_z = lax.axis_index("z")
        my_id = my_x * 4 + my_y * 2 + my_z
        start = (2 * my_x + my_z) * rows

        cps = []
        for c in range(n_chunks):
            cpx = pltpu.make_async_copy(
                x_hbm.at[pl.ds(start + c * half, half), :],
                xbuf.at[c], local_sems.at[2 * c])
            cpd = pltpu.make_async_copy(
                dy_hbm.at[pl.ds(start + c * half, half), :],
                dybuf.at[c], local_sems.at[2 * c + 1])
            cpx.start()
            cpd.start()
            cps.append((cpx, cpd))

        barrier_sem = pltpu.get_barrier_semaphore()
        peers = []
        for mask in range(1, _NDEV):
            peer = (my_x ^ (mask >> 2), my_y ^ ((mask >> 1) & 1),
                    my_z ^ (mask & 1))
            peers.append(peer)
            pl.semaphore_signal(
                barrier_sem, inc=1, device_id=peer,
                device_id_type=pl.DeviceIdType.MESH,
            )

        pdg = None
        pdb = None
        for c in range(n_chunks):
            cps[c][0].wait()
            xv = xbuf[c]
            mu = jnp.mean(xv, axis=1, keepdims=True)
            var = jnp.mean(xv * xv, axis=1, keepdims=True) - mu * mu
            rstd = lax.rsqrt(var + 1e-5)
            xhat = (xv - mu) * rstd
            cps[c][1].wait()
            dyv = dybuf[c]
            g = jnp.sum(dyv * xhat, axis=0)
            b = jnp.sum(dyv, axis=0)
            pdg = g if pdg is None else pdg + g
            pdb = b if pdb is None else pdb + b
        gather_ref[my_id, 0, :] = pdg
        gather_ref[my_id, 1, :] = pdb

        pl.semaphore_wait(barrier_sem, _NDEV - 1)

        rdmas = []
        for k, peer in enumerate(peers):
            rdma = pltpu.make_async_remote_copy(
                src_ref=gather_ref.at[my_id],
                dst_ref=gather_ref.at[my_id],
                send_sem=send_sems.at[k],
                recv_sem=recv_sems.at[k],
                device_id=peer,
                device_id_type=pl.DeviceIdType.MESH,
            )
            rdma.start()
            rdmas.append(rdma)
        for rdma in rdmas:
            rdma.wait()

        out_ref[:, :] = jnp.sum(gather_ref[:, :, :], axis=0)

    return pl.pallas_call(
        body,
        out_shape=jax.ShapeDtypeStruct((2, d), jnp.float32),
        in_specs=[
            pl.BlockSpec(memory_space=pltpu.MemorySpace.HBM),
            pl.BlockSpec(memory_space=pltpu.MemorySpace.HBM),
        ],
        out_specs=pl.BlockSpec(memory_space=pltpu.VMEM),
        scratch_shapes=[
            pltpu.VMEM((n_chunks, half, d), jnp.float32),
            pltpu.VMEM((n_chunks, half, d), jnp.float32),
            pltpu.VMEM((_NDEV, 2, d), jnp.float32),
            pltpu.SemaphoreType.DMA((2 * n_chunks,)),
            pltpu.SemaphoreType.DMA((_NDEV - 1,)),
            pltpu.SemaphoreType.DMA((_NDEV - 1,)),
        ],
        compiler_params=pltpu.CompilerParams(collective_id=0),
    )(
        pltpu.with_memory_space_constraint(x, pltpu.MemorySpace.HBM),
        pltpu.with_memory_space_constraint(dy, pltpu.MemorySpace.HBM),
    )
